# baseline (speedup 1.0000x reference)
import jax
import jax.numpy as jnp
from jax import lax
from jax.experimental import pallas as pl
from jax.experimental.pallas import tpu as pltpu
from jax.experimental.pallas import tpu_sc as plsc

NC = 2
NS = 16
NW = NC * NS
LANES = 16

CHUNK = 32
GATHER_IDX = 64

PER_W = (2880, 320)


def _sc_bag_kernel(B_pad, ns, d, fs):
    assert B_pad == NS * (PER_W[0] + PER_W[1])
    n_chunks_c = (PER_W[0] // CHUNK, PER_W[1] // CHUNK)
    assert all(n % 2 == 0 for n in n_chunks_c)
    idx_per_chunk = CHUNK * ns
    n_gathers = idx_per_chunk // GATHER_IDX
    d_regs = d // LANES

    mesh = plsc.VectorSubcoreMesh(
        core_axis_name="c", subcore_axis_name="s",
        num_cores=NC, num_subcores=NS)

    def body(table_hbm, idx_hbm, u_hbm, ut_hbm, w_hbm, a_hbm, out_hbm,
             idx_v0, idx_v1, rows_v0, rows_v1, out_v0, out_v1,
             u_v, ut_v, w_v, a_v,
             gsem0, gsem1, isem0, isem1, osem0, osem1):
        idx_b = (idx_v0, idx_v1)
        rows_b = (rows_v0, rows_v1)
        out_b = (out_v0, out_v1)
        gsem = (gsem0, gsem1)
        isem = (isem0, isem1)
        osem = (osem0, osem1)

        pltpu.sync_copy(u_hbm, u_v)
        pltpu.sync_copy(ut_hbm, ut_v)
        pltpu.sync_copy(w_hbm, w_v)
        pltpu.sync_copy(a_hbm, a_v)
        a_vec = a_v[...]
        t = jnp.zeros((LANES,), dtype=jnp.float32)
        for j in range(fs):
            t = t + a_vec[j] * u_v[j, :]
        m = w_v[...] * t
        v = jnp.zeros((LANES,), dtype=jnp.float32)
        for j in range(fs):
            v = v + m[j] * ut_v[j, :]
        vk = [v[k + 1] for k in range(ns)]

        c = lax.axis_index("c")
        s = lax.axis_index("s")
        on_c0 = c == 0
        row_base = jnp.where(on_c0, s * PER_W[0],
                             NS * PER_W[0] + s * PER_W[1])
        n_chunks = jnp.where(on_c0, n_chunks_c[0], n_chunks_c[1])

        def stage_idx(ci, b):
            pltpu.async_copy(
                idx_hbm.at[pl.ds((row_base + ci * CHUNK) * ns, idx_per_chunk)],
                idx_b[b], isem[b])

        def wait_idx(b):
            pltpu.make_async_copy(
                idx_hbm.at[pl.ds(0, idx_per_chunk)], idx_b[b], isem[b]
            ).wait()

        def fire_gathers(b):
            for g in range(n_gathers):
                sl = pl.ds(g * GATHER_IDX, GATHER_IDX)
                pltpu.async_copy(table_hbm.at[idx_b[b].at[sl]],
                                 rows_b[b].at[sl], gsem[b])

        def drain_gathers(b):
            pltpu.make_async_copy(
                table_hbm.at[pl.ds(0, idx_per_chunk)], rows_b[b], gsem[b]
            ).wait()

        def drain_out(b):
            pltpu.make_async_copy(
                out_hbm.at[pl.ds(0, CHUNK)], out_b[b], osem[b]
            ).wait()

        def compute(ci, b):
            rows = rows_b[b]
            out = out_b[b]

            @plsc.parallel_loop(0, CHUNK, unroll=2)
            def _(r):
                base = r * ns
                for j in range(d_regs):
                    sl = pl.ds(j * LANES, LANES)
                    acc = vk[0] * rows[base, sl]
                    for k in range(1, ns):
                        acc = acc + vk[k] * rows[base + k, sl]
                    out[r, sl] = acc

            pltpu.async_copy(
                out, out_hbm.at[pl.ds(row_base + ci * CHUNK, CHUNK)], osem[b])

        @pl.when(n_chunks > 0)
        def _():
            stage_idx(0, 0)
            wait_idx(0)
            fire_gathers(0)
            stage_idx(1, 1)

        def pair_body(i, _):
            c0 = 2 * i
            wait_idx(1)
            fire_gathers(1)
            drain_gathers(0)

            @pl.when(c0 + 2 < n_chunks)
            def _():
                stage_idx(c0 + 2, 0)

            @pl.when(i > 0)
            def _():
                drain_out(0)

            compute(c0, 0)

            @pl.when(c0 + 2 < n_chunks)
            def _():
                wait_idx(0)
                fire_gathers(0)

            drain_gathers(1)

            @pl.when(c0 + 3 < n_chunks)
            def _():
                stage_idx(c0 + 3, 1)

            @pl.when(i > 0)
            def _():
                drain_out(1)

            compute(c0 + 1, 1)
            return 0

        lax.fori_loop(0, n_chunks // 2, pair_body, 0, unroll=False)

        @pl.when(n_chunks > 0)
        def _():
            drain_out(0)
            drain_out(1)

    return pl.kernel(
        body,
        out_type=jax.ShapeDtypeStruct((B_pad, d), jnp.float32),
        mesh=mesh,
        scratch_types=[
            pltpu.VMEM((idx_per_chunk,), jnp.int32),
            pltpu.VMEM((idx_per_chunk,), jnp.int32),
            pltpu.VMEM((idx_per_chunk, d), jnp.float32),
            pltpu.VMEM((idx_per_chunk, d), jnp.float32),
            pltpu.VMEM((CHUNK, d), jnp.float32),
            pltpu.VMEM((CHUNK, d), jnp.float32),
            pltpu.VMEM((LANES, LANES), jnp.float32),
            pltpu.VMEM((LANES, LANES), jnp.float32),
            pltpu.VMEM((LANES,), jnp.float32),
            pltpu.VMEM((LANES,), jnp.float32),
            pltpu.SemaphoreType.DMA,
            pltpu.SemaphoreType.DMA,
            pltpu.SemaphoreType.DMA,
            pltpu.SemaphoreType.DMA,
            pltpu.SemaphoreType.DMA,
            pltpu.SemaphoreType.DMA,
        ],
    )


def kernel(neighbor_idx, feat_table, U, weight, avgweight):
    B, ns = neighbor_idx.shape
    fs = ns + 2
    d = feat_table.shape[1]

    B_pad = NS * (PER_W[0] + PER_W[1])
    assert B_pad >= B

    idx_flat = neighbor_idx.astype(jnp.int32).reshape(-1)
    idx_flat = jnp.pad(idx_flat, (0, B_pad * ns - B * ns))

    u_pad = jnp.zeros((LANES, LANES), jnp.float32).at[:fs, :fs].set(U)
    ut_pad = jnp.zeros((LANES, LANES), jnp.float32).at[:fs, :fs].set(U.T)
    w_pad = jnp.zeros((LANES,), jnp.float32).at[:fs].set(weight.reshape(-1))
    a_pad = jnp.zeros((LANES,), jnp.float32).at[:fs].set(avgweight.reshape(-1))

    out = _sc_bag_kernel(B_pad, ns, d, fs)(
        feat_table, idx_flat, u_pad, ut_pad, w_pad, a_pad)
    return out[:B]


# speedup vs baseline: 3.6198x; 3.6198x over previous
import jax
import jax.numpy as jnp
from jax import lax
from jax.experimental import pallas as pl
from jax.experimental.pallas import tpu as pltpu
from jax.experimental.pallas import tpu_sc as plsc

NC = 2
NS = 16
NW = NC * NS
LANES = 16

CHUNK = 32
GATHER_IDX = 64

PER_W = (1600, 1600)


def _sc_bag_kernel(B_pad, ns, d, fs):
    assert B_pad == NS * (PER_W[0] + PER_W[1])
    n_chunks_c = (PER_W[0] // CHUNK, PER_W[1] // CHUNK)
    assert all(n % 2 == 0 for n in n_chunks_c)
    idx_per_chunk = CHUNK * ns
    n_gathers = idx_per_chunk // GATHER_IDX
    d_regs = d // LANES

    mesh = plsc.VectorSubcoreMesh(
        core_axis_name="c", subcore_axis_name="s",
        num_cores=NC, num_subcores=NS)

    def body(table_hbm, idx_hbm, u_hbm, ut_hbm, w_hbm, a_hbm, out_hbm,
             idx_v0, idx_v1, rows_v0, rows_v1, out_v0, out_v1,
             u_v, ut_v, w_v, a_v,
             gsem0, gsem1, isem0, isem1, osem0, osem1):
        idx_b = (idx_v0, idx_v1)
        rows_b = (rows_v0, rows_v1)
        out_b = (out_v0, out_v1)
        gsem = (gsem0, gsem1)
        isem = (isem0, isem1)
        osem = (osem0, osem1)

        pltpu.sync_copy(u_hbm, u_v)
        pltpu.sync_copy(ut_hbm, ut_v)
        pltpu.sync_copy(w_hbm, w_v)
        pltpu.sync_copy(a_hbm, a_v)
        a_vec = a_v[...]
        t = jnp.zeros((LANES,), dtype=jnp.float32)
        for j in range(fs):
            t = t + a_vec[j] * u_v[j, :]
        m = w_v[...] * t
        v = jnp.zeros((LANES,), dtype=jnp.float32)
        for j in range(fs):
            v = v + m[j] * ut_v[j, :]
        vk = [v[k + 1] for k in range(ns)]

        c = lax.axis_index("c")
        s = lax.axis_index("s")
        on_c0 = c == 0
        row_base = jnp.where(on_c0, s * PER_W[0],
                             NS * PER_W[0] + s * PER_W[1])
        n_chunks = jnp.where(on_c0, n_chunks_c[0], n_chunks_c[1])

        def stage_idx(ci, b):
            pltpu.async_copy(
                idx_hbm.at[pl.ds((row_base + ci * CHUNK) * ns, idx_per_chunk)],
                idx_b[b], isem[b])

        def wait_idx(b):
            pltpu.make_async_copy(
                idx_hbm.at[pl.ds(0, idx_per_chunk)], idx_b[b], isem[b]
            ).wait()

        def fire_gathers(b):
            for g in range(n_gathers):
                sl = pl.ds(g * GATHER_IDX, GATHER_IDX)
                pltpu.async_copy(table_hbm.at[idx_b[b].at[sl]],
                                 rows_b[b].at[sl], gsem[b])

        def drain_gathers(b):
            pltpu.make_async_copy(
                table_hbm.at[pl.ds(0, idx_per_chunk)], rows_b[b], gsem[b]
            ).wait()

        def drain_out(b):
            pltpu.make_async_copy(
                out_hbm.at[pl.ds(0, CHUNK)], out_b[b], osem[b]
            ).wait()

        def compute(ci, b):
            rows = rows_b[b]
            out = out_b[b]

            @plsc.parallel_loop(0, CHUNK, unroll=2)
            def _(r):
                base = r * ns
                for j in range(d_regs):
                    sl = pl.ds(j * LANES, LANES)
                    acc = vk[0] * rows[base, sl]
                    for k in range(1, ns):
                        acc = acc + vk[k] * rows[base + k, sl]
                    out[r, sl] = acc

            pltpu.async_copy(
                out, out_hbm.at[pl.ds(row_base + ci * CHUNK, CHUNK)], osem[b])

        @pl.when(n_chunks > 0)
        def _():
            stage_idx(0, 0)
            wait_idx(0)
            fire_gathers(0)
            stage_idx(1, 1)

        def pair_body(i, _):
            c0 = 2 * i
            wait_idx(1)
            fire_gathers(1)
            drain_gathers(0)

            @pl.when(c0 + 2 < n_chunks)
            def _():
                stage_idx(c0 + 2, 0)

            @pl.when(i > 0)
            def _():
                drain_out(0)

            compute(c0, 0)

            @pl.when(c0 + 2 < n_chunks)
            def _():
                wait_idx(0)
                fire_gathers(0)

            drain_gathers(1)

            @pl.when(c0 + 3 < n_chunks)
            def _():
                stage_idx(c0 + 3, 1)

            @pl.when(i > 0)
            def _():
                drain_out(1)

            compute(c0 + 1, 1)
            return 0

        lax.fori_loop(0, n_chunks // 2, pair_body, 0, unroll=False)

        @pl.when(n_chunks > 0)
        def _():
            drain_out(0)
            drain_out(1)

    return pl.kernel(
        body,
        out_type=jax.ShapeDtypeStruct((B_pad, d), jnp.float32),
        mesh=mesh,
        scratch_types=[
            pltpu.VMEM((idx_per_chunk,), jnp.int32),
            pltpu.VMEM((idx_per_chunk,), jnp.int32),
            pltpu.VMEM((idx_per_chunk, d), jnp.float32),
            pltpu.VMEM((idx_per_chunk, d), jnp.float32),
            pltpu.VMEM((CHUNK, d), jnp.float32),
            pltpu.VMEM((CHUNK, d), jnp.float32),
            pltpu.VMEM((LANES, LANES), jnp.float32),
            pltpu.VMEM((LANES, LANES), jnp.float32),
            pltpu.VMEM((LANES,), jnp.float32),
            pltpu.VMEM((LANES,), jnp.float32),
            pltpu.SemaphoreType.DMA,
            pltpu.SemaphoreType.DMA,
            pltpu.SemaphoreType.DMA,
            pltpu.SemaphoreType.DMA,
            pltpu.SemaphoreType.DMA,
            pltpu.SemaphoreType.DMA,
        ],
    )


def kernel(neighbor_idx, feat_table, U, weight, avgweight):
    B, ns = neighbor_idx.shape
    fs = ns + 2
    d = feat_table.shape[1]

    B_pad = NS * (PER_W[0] + PER_W[1])
    assert B_pad >= B

    idx_flat = neighbor_idx.astype(jnp.int32).reshape(-1)
    pad_n = B_pad * ns - B * ns
    N_rows = feat_table.shape[0]
    pad_idx = (jnp.arange(pad_n, dtype=jnp.int32) * 769) % N_rows
    idx_flat = jnp.concatenate([idx_flat, pad_idx])

    u_pad = jnp.zeros((LANES, LANES), jnp.float32).at[:fs, :fs].set(U)
    ut_pad = jnp.zeros((LANES, LANES), jnp.float32).at[:fs, :fs].set(U.T)
    w_pad = jnp.zeros((LANES,), jnp.float32).at[:fs].set(weight.reshape(-1))
    a_pad = jnp.zeros((LANES,), jnp.float32).at[:fs].set(avgweight.reshape(-1))

    out = _sc_bag_kernel(B_pad, ns, d, fs)(
        feat_table, idx_flat, u_pad, ut_pad, w_pad, a_pad)
    return out[:B]
